# initial kernel scaffold (unmeasured)
import jax
import jax.numpy as jnp
from jax import lax
from jax.experimental import pallas as pl
from jax.experimental.pallas import tpu as pltpu

N_DEV = 4
B, SQ, D = 2, 256, 768
HQ_LOC, DH = 8, 64
SKV = 512
DQ_LOC = HQ_LOC * DH


def kernel(x, Wq, Wo, K_ext, V_ext):
    def body(x_ref, wq_ref, wo_ref, k_ref, v_ref, out_ref,
             comm_ref, attn_ref, send_sems, recv_sems):
        my = lax.axis_index("i")

        wq = wq_ref[...].astype(jnp.bfloat16)
        wo = wo_ref[...].astype(jnp.bfloat16)
        for b in range(B):
            xb = x_ref[b].astype(jnp.bfloat16)
            qb = jnp.dot(xb, wq, preferred_element_type=jnp.float32)
            qb = qb * 0.125
            for h in range(HQ_LOC):
                q = qb[:, h * DH:(h + 1) * DH].astype(jnp.bfloat16)
                k = k_ref[b, :, h, :].astype(jnp.bfloat16)
                v = v_ref[b, :, h, :].astype(jnp.bfloat16)
                s = lax.dot_general(q, k, (((1,), (1,)), ((), ())),
                                    preferred_element_type=jnp.float32)
                m = jnp.max(s, axis=-1, keepdims=True)
                p = jnp.exp(s - m)
                denom = jnp.sum(p, axis=-1, keepdims=True)
                o = jnp.dot((p / denom).astype(jnp.bfloat16), v,
                            preferred_element_type=jnp.float32)
                attn_ref[:, h * DH:(h + 1) * DH] = o.astype(jnp.bfloat16)
            out_ref[b] = jnp.dot(attn_ref[...], wo,
                                 preferred_element_type=jnp.float32)

        peers = [jnp.bitwise_xor(my, 1), 3 - my]
        for s in range(2):
            rdma = pltpu.make_async_remote_copy(
                src_ref=out_ref,
                dst_ref=comm_ref.at[s],
                send_sem=send_sems.at[s],
                recv_sem=recv_sems.at[s],
                device_id=(peers[s],),
                device_id_type=pl.DeviceIdType.MESH,
            )
            rdma.start()
            rdma.wait()
            out_ref[...] = out_ref[...] + comm_ref[s]

    return pl.pallas_call(
        body,
        out_shape=jax.ShapeDtypeStruct((B, SQ, D), jnp.float32),
        in_specs=[pl.BlockSpec(memory_space=pltpu.VMEM)] * 5,
        out_specs=pl.BlockSpec(memory_space=pltpu.VMEM),
        scratch_shapes=[
            pltpu.VMEM((2, B, SQ, D), jnp.float32),
            pltpu.VMEM((SQ, DQ_LOC), jnp.bfloat16),
            pltpu.SemaphoreType.DMA((2,)),
            pltpu.SemaphoreType.DMA((2,)),
        ],
        compiler_params=pltpu.CompilerParams(collective_id=0),
    )(x, Wq, Wo, K_ext, V_ext)


# baseline (device time: 65633 ns/iter reference)
import jax
import jax.numpy as jnp
from jax import lax
from jax.experimental import pallas as pl
from jax.experimental.pallas import tpu as pltpu

N_DEV = 4
B, SQ, D = 2, 256, 768
HQ_LOC, DH = 8, 64
SKV = 512
DQ_LOC = HQ_LOC * DH


def kernel(x, Wq, Wo, K_ext, V_ext):
    def body(x_ref, wq_ref, wo_ref, k_ref, v_ref, out_ref,
             comm_ref, attn_ref, send_sems, recv_sems):
        my = lax.axis_index("i")

        wq = wq_ref[...].astype(jnp.bfloat16)
        wo = wo_ref[...].astype(jnp.bfloat16)
        for b in range(B):
            xb = x_ref[b].astype(jnp.bfloat16)
            qb = jnp.dot(xb, wq, preferred_element_type=jnp.float32)
            qb = qb * 0.125
            for h in range(HQ_LOC):
                q = qb[:, h * DH:(h + 1) * DH].astype(jnp.bfloat16)
                k = k_ref[b, :, h, :].astype(jnp.bfloat16)
                v = v_ref[b, :, h, :].astype(jnp.bfloat16)
                s = lax.dot_general(q, k, (((1,), (1,)), ((), ())),
                                    preferred_element_type=jnp.float32)
                m = jnp.max(s, axis=-1, keepdims=True)
                p = jnp.exp(s - m)
                denom = jnp.sum(p, axis=-1, keepdims=True)
                o = jnp.dot((p / denom).astype(jnp.bfloat16), v,
                            preferred_element_type=jnp.float32)
                attn_ref[:, h * DH:(h + 1) * DH] = o.astype(jnp.bfloat16)
            out_ref[b] = jnp.dot(attn_ref[...], wo,
                                 preferred_element_type=jnp.float32)

        peers = [jnp.bitwise_xor(my, 1), 3 - my]
        for s in range(2):
            rdma = pltpu.make_async_remote_copy(
                src_ref=out_ref,
                dst_ref=comm_ref.at[s],
                send_sem=send_sems.at[s],
                recv_sem=recv_sems.at[s],
                device_id=(peers[s],),
                device_id_type=pl.DeviceIdType.MESH,
            )
            rdma.start()
            rdma.wait()
            out_ref[...] = out_ref[...] + comm_ref[s]

    return pl.pallas_call(
        body,
        out_shape=jax.ShapeDtypeStruct((B, SQ, D), jnp.float32),
        in_specs=[pl.BlockSpec(memory_space=pltpu.VMEM)] * 5,
        out_specs=pl.BlockSpec(memory_space=pltpu.VMEM),
        scratch_shapes=[
            pltpu.VMEM((2, B, SQ, D), jnp.float32),
            pltpu.VMEM((SQ, DQ_LOC), jnp.bfloat16),
            pltpu.SemaphoreType.DMA((2,)),
            pltpu.SemaphoreType.DMA((2,)),
        ],
    )(x, Wq, Wo, K_ext, V_ext)


# device time: 47470 ns/iter; 1.3826x vs baseline; 1.3826x over previous
import jax
import jax.numpy as jnp
from jax import lax
from jax.experimental import pallas as pl
from jax.experimental.pallas import tpu as pltpu

N_DEV = 4
B, SQ, D = 2, 256, 768
HQ_LOC, DH = 8, 64
SKV = 512
DQ_LOC = HQ_LOC * DH


def kernel(x, Wq, Wo, K_ext, V_ext):
    def body(x_ref, wq_ref, wo_ref, k_ref, v_ref, out_ref,
             send_ref, recv_ref, attn_ref, send_sems, recv_sems):
        my = lax.axis_index("i")

        wq = wq_ref[...].astype(jnp.bfloat16)
        wo = wo_ref[...].astype(jnp.bfloat16)
        for b in range(B):
            xb = x_ref[b].astype(jnp.bfloat16)
            qb = jnp.dot(xb, wq, preferred_element_type=jnp.float32)
            qb = qb * 0.125
            for h in range(HQ_LOC):
                q = qb[:, h * DH:(h + 1) * DH].astype(jnp.bfloat16)
                k = k_ref[b, :, h, :].astype(jnp.bfloat16)
                v = v_ref[b, :, h, :].astype(jnp.bfloat16)
                s = lax.dot_general(q, k, (((1,), (1,)), ((), ())),
                                    preferred_element_type=jnp.float32)
                m = jnp.max(s, axis=-1, keepdims=True)
                p = jnp.exp(s - m)
                denom = jnp.sum(p, axis=-1, keepdims=True)
                o = jnp.dot(p.astype(jnp.bfloat16), v,
                            preferred_element_type=jnp.float32)
                o = o * (1.0 / denom)
                attn_ref[:, h * DH:(h + 1) * DH] = o.astype(jnp.bfloat16)
            send_ref[0, b] = jnp.dot(attn_ref[...], wo,
                                     preferred_element_type=jnp.float32
                                     ).astype(jnp.bfloat16)

        peers = [jnp.bitwise_xor(my, 1), 3 - my]
        for s in range(2):
            rdma = pltpu.make_async_remote_copy(
                src_ref=send_ref.at[s],
                dst_ref=recv_ref.at[s],
                send_sem=send_sems.at[s],
                recv_sem=recv_sems.at[s],
                device_id=(peers[s],),
                device_id_type=pl.DeviceIdType.MESH,
            )
            rdma.start()
            rdma.wait()
            acc = (send_ref[s].astype(jnp.float32)
                   + recv_ref[s].astype(jnp.float32))
            if s == 0:
                send_ref[1] = acc.astype(jnp.bfloat16)
            else:
                out_ref[...] = acc

    return pl.pallas_call(
        body,
        out_shape=jax.ShapeDtypeStruct((B, SQ, D), jnp.float32),
        in_specs=[pl.BlockSpec(memory_space=pltpu.VMEM)] * 5,
        out_specs=pl.BlockSpec(memory_space=pltpu.VMEM),
        scratch_shapes=[
            pltpu.VMEM((2, B, SQ, D), jnp.bfloat16),
            pltpu.VMEM((2, B, SQ, D), jnp.bfloat16),
            pltpu.VMEM((SQ, DQ_LOC), jnp.bfloat16),
            pltpu.SemaphoreType.DMA((2,)),
            pltpu.SemaphoreType.DMA((2,)),
        ],
    )(x, Wq, Wo, K_ext, V_ext)


# device time: 39212 ns/iter; 1.6738x vs baseline; 1.2106x over previous
import jax
import jax.numpy as jnp
from jax import lax
from jax.experimental import pallas as pl
from jax.experimental.pallas import tpu as pltpu

N_DEV = 4
B, SQ, D = 2, 256, 768
HQ_LOC, DH = 8, 64
SKV = 512
DQ_LOC = HQ_LOC * DH


def kernel(x, Wq, Wo, K_ext, V_ext):
    def body(x_ref, wq_ref, wo_ref, k_ref, v_ref, out_ref,
             send_ref, recv_ref, attn_ref, send_sems, recv_sems):
        my = lax.axis_index("i")
        peers = [jnp.bitwise_xor(my, 1), 3 - my]

        wq = wq_ref[...].astype(jnp.bfloat16)
        wo = wo_ref[...].astype(jnp.bfloat16)

        def partial_out(b):
            xb = x_ref[b].astype(jnp.bfloat16)
            qb = jnp.dot(xb, wq, preferred_element_type=jnp.float32)
            qb = qb * 0.125
            for h in range(HQ_LOC):
                q = qb[:, h * DH:(h + 1) * DH].astype(jnp.bfloat16)
                k = k_ref[b, :, h, :].astype(jnp.bfloat16)
                v = v_ref[b, :, h, :].astype(jnp.bfloat16)
                s = lax.dot_general(q, k, (((1,), (1,)), ((), ())),
                                    preferred_element_type=jnp.float32)
                m = jnp.max(s, axis=-1, keepdims=True)
                p = jnp.exp(s - m)
                denom = jnp.sum(p, axis=-1, keepdims=True)
                o = jnp.dot(p.astype(jnp.bfloat16), v,
                            preferred_element_type=jnp.float32)
                o = o * (1.0 / denom)
                attn_ref[:, h * DH:(h + 1) * DH] = o.astype(jnp.bfloat16)
            return jnp.dot(attn_ref[...], wo,
                           preferred_element_type=jnp.float32)

        def exchange(slot, stage):
            rdma = pltpu.make_async_remote_copy(
                src_ref=send_ref.at[slot],
                dst_ref=recv_ref.at[slot],
                send_sem=send_sems.at[slot],
                recv_sem=recv_sems.at[slot],
                device_id=(peers[stage],),
                device_id_type=pl.DeviceIdType.MESH,
            )
            rdma.start()
            return rdma

        s0 = []
        for b in range(B):
            send_ref[b] = partial_out(b).astype(jnp.bfloat16)
            s0.append(exchange(b, 0))

        s1 = []
        for b in range(B):
            s0[b].wait()
            acc = (send_ref[b].astype(jnp.float32)
                   + recv_ref[b].astype(jnp.float32))
            send_ref[2 + b] = acc.astype(jnp.bfloat16)
            s1.append(exchange(2 + b, 1))

        for b in range(B):
            s1[b].wait()
            out_ref[b] = (send_ref[2 + b].astype(jnp.float32)
                          + recv_ref[2 + b].astype(jnp.float32))

    return pl.pallas_call(
        body,
        out_shape=jax.ShapeDtypeStruct((B, SQ, D), jnp.float32),
        in_specs=[pl.BlockSpec(memory_space=pltpu.VMEM)] * 5,
        out_specs=pl.BlockSpec(memory_space=pltpu.VMEM),
        scratch_shapes=[
            pltpu.VMEM((4, SQ, D), jnp.bfloat16),
            pltpu.VMEM((4, SQ, D), jnp.bfloat16),
            pltpu.VMEM((SQ, DQ_LOC), jnp.bfloat16),
            pltpu.SemaphoreType.DMA((4,)),
            pltpu.SemaphoreType.DMA((4,)),
        ],
    )(x, Wq, Wo, K_ext, V_ext)


# device time: 30459 ns/iter; 2.1548x vs baseline; 1.2874x over previous
import jax
import jax.numpy as jnp
from jax import lax
from jax.experimental import pallas as pl
from jax.experimental.pallas import tpu as pltpu

N_DEV = 4
B, SQ, D = 2, 256, 768
HQ_LOC, DH = 8, 64
SKV = 512
DQ_LOC = HQ_LOC * DH


def kernel(x, Wq, Wo, K_ext, V_ext):
    def body(x_ref, wq_ref, wo_ref, k_ref, v_ref, out_ref,
             send_ref, recv_ref, attn_ref, send_sems, recv_sems):
        my = lax.axis_index("i")
        peers = [jnp.bitwise_xor(my, 1), 3 - my]

        wq = wq_ref[...].astype(jnp.bfloat16)
        wo = wo_ref[...].astype(jnp.bfloat16)

        def partial_out(b):
            xb = x_ref[b].astype(jnp.bfloat16)
            qb = jnp.dot(xb, wq, preferred_element_type=jnp.float32)
            qb = qb * 0.125
            kb = k_ref[b].astype(jnp.bfloat16)
            vb = v_ref[b].astype(jnp.bfloat16)
            for h in range(HQ_LOC):
                q = qb[:, h * DH:(h + 1) * DH].astype(jnp.bfloat16)
                k = kb[:, h * DH:(h + 1) * DH]
                v = vb[:, h * DH:(h + 1) * DH]
                s = lax.dot_general(q, k, (((1,), (1,)), ((), ())),
                                    preferred_element_type=jnp.float32)
                p = jnp.exp(s)
                denom = jnp.sum(p, axis=-1, keepdims=True)
                o = jnp.dot(p.astype(jnp.bfloat16), v,
                            preferred_element_type=jnp.float32)
                o = o * (1.0 / denom)
                attn_ref[:, h * DH:(h + 1) * DH] = o.astype(jnp.bfloat16)
            return jnp.dot(attn_ref[...], wo,
                           preferred_element_type=jnp.float32)

        def exchange(slot, stage):
            rdma = pltpu.make_async_remote_copy(
                src_ref=send_ref.at[slot],
                dst_ref=recv_ref.at[slot],
                send_sem=send_sems.at[slot],
                recv_sem=recv_sems.at[slot],
                device_id=(peers[stage],),
                device_id_type=pl.DeviceIdType.MESH,
            )
            rdma.start()
            return rdma

        s0 = []
        for b in range(B):
            send_ref[b] = partial_out(b).astype(jnp.bfloat16)
            s0.append(exchange(b, 0))

        s1 = []
        for b in range(B):
            s0[b].wait()
            acc = (send_ref[b].astype(jnp.float32)
                   + recv_ref[b].astype(jnp.float32))
            send_ref[2 + b] = acc.astype(jnp.bfloat16)
            s1.append(exchange(2 + b, 1))

        for b in range(B):
            s1[b].wait()
            out_ref[b] = (send_ref[2 + b].astype(jnp.float32)
                          + recv_ref[2 + b].astype(jnp.float32))

    K_t = K_ext.reshape(B, SKV, DQ_LOC)
    V_t = V_ext.reshape(B, SKV, DQ_LOC)
    return pl.pallas_call(
        body,
        out_shape=jax.ShapeDtypeStruct((B, SQ, D), jnp.float32),
        in_specs=[pl.BlockSpec(memory_space=pltpu.VMEM)] * 5,
        out_specs=pl.BlockSpec(memory_space=pltpu.VMEM),
        scratch_shapes=[
            pltpu.VMEM((4, SQ, D), jnp.bfloat16),
            pltpu.VMEM((4, SQ, D), jnp.bfloat16),
            pltpu.VMEM((SQ, DQ_LOC), jnp.bfloat16),
            pltpu.SemaphoreType.DMA((4,)),
            pltpu.SemaphoreType.DMA((4,)),
        ],
    )(x, Wq, Wo, K_t, V_t)


# device time: 26892 ns/iter; 2.4406x vs baseline; 1.1326x over previous
import jax
import jax.numpy as jnp
from jax import lax
from jax.experimental import pallas as pl
from jax.experimental.pallas import tpu as pltpu

N_DEV = 4
B, SQ, D = 2, 256, 768
HQ_LOC, DH = 8, 64
SKV = 512
DQ_LOC = HQ_LOC * DH


def kernel(x, Wq, Wo, K_ext, V_ext):
    def body(x_ref, wq_ref, wo_ref, k_ref, v_ref, out_ref,
             send_ref, recv_ref, attn_ref, send_sems, recv_sems):
        my = lax.axis_index("i")
        peers = [jnp.bitwise_xor(my, 1), 3 - my]

        barrier_sem = pltpu.get_barrier_semaphore()
        for stage in range(2):
            pl.semaphore_signal(barrier_sem, inc=1, device_id=(peers[stage],),
                                device_id_type=pl.DeviceIdType.MESH)
        pl.semaphore_wait(barrier_sem, 2)

        wq = wq_ref[...].astype(jnp.bfloat16)
        wo = wo_ref[...].astype(jnp.bfloat16)

        def partial_out(b):
            xb = x_ref[b].astype(jnp.bfloat16)
            qb = jnp.dot(xb, wq, preferred_element_type=jnp.float32)
            qb = (qb * 0.125).astype(jnp.bfloat16)
            kb = k_ref[b].astype(jnp.bfloat16)
            vb = v_ref[b].astype(jnp.bfloat16)
            for h in range(HQ_LOC):
                q = qb[:, h * DH:(h + 1) * DH]
                k = kb[:, h * DH:(h + 1) * DH]
                v = vb[:, h * DH:(h + 1) * DH]
                s = lax.dot_general(q, k, (((1,), (1,)), ((), ())),
                                    preferred_element_type=jnp.float32)
                p = jnp.exp(s)
                denom = jnp.sum(p, axis=-1, keepdims=True)
                o = jnp.dot(p.astype(jnp.bfloat16), v,
                            preferred_element_type=jnp.float32)
                o = o * (1.0 / denom)
                attn_ref[:, h * DH:(h + 1) * DH] = o.astype(jnp.bfloat16)
            return jnp.dot(attn_ref[...], wo,
                           preferred_element_type=jnp.float32
                           ).astype(jnp.bfloat16)

        def exchange(slot, stage):
            rdma = pltpu.make_async_remote_copy(
                src_ref=send_ref.at[slot],
                dst_ref=recv_ref.at[slot],
                send_sem=send_sems.at[slot],
                recv_sem=recv_sems.at[slot],
                device_id=(peers[stage],),
                device_id_type=pl.DeviceIdType.MESH,
            )
            rdma.start()
            return rdma

        s0 = []
        for b in range(B):
            send_ref[b] = partial_out(b)
            s0.append(exchange(b, 0))

        s1 = []
        for b in range(B):
            s0[b].wait()
            send_ref[2 + b] = send_ref[b] + recv_ref[b]
            s1.append(exchange(2 + b, 1))

        for b in range(B):
            s1[b].wait()
            out_ref[b] = (send_ref[2 + b].astype(jnp.float32)
                          + recv_ref[2 + b].astype(jnp.float32))

    K_t = K_ext.reshape(B, SKV, DQ_LOC)
    V_t = V_ext.reshape(B, SKV, DQ_LOC)
    return pl.pallas_call(
        body,
        out_shape=jax.ShapeDtypeStruct((B, SQ, D), jnp.float32),
        in_specs=[pl.BlockSpec(memory_space=pltpu.VMEM)] * 5,
        out_specs=pl.BlockSpec(memory_space=pltpu.VMEM),
        scratch_shapes=[
            pltpu.VMEM((4, SQ, D), jnp.bfloat16),
            pltpu.VMEM((4, SQ, D), jnp.bfloat16),
            pltpu.VMEM((SQ, DQ_LOC), jnp.bfloat16),
            pltpu.SemaphoreType.DMA((4,)),
            pltpu.SemaphoreType.DMA((4,)),
        ],
        compiler_params=pltpu.CompilerParams(collective_id=0),
    )(x, Wq, Wo, K_t, V_t)


# device time: 25059 ns/iter; 2.6191x vs baseline; 1.0731x over previous
import jax
import jax.numpy as jnp
from jax import lax
from jax.experimental import pallas as pl
from jax.experimental.pallas import tpu as pltpu

N_DEV = 4
B, SQ, D = 2, 256, 768
HQ_LOC, DH = 8, 64
SKV = 512
DQ_LOC = HQ_LOC * DH
HD = D // 2
NC = 2 * B


def kernel(x, Wq, Wo, K_ext, V_ext):
    def body(x_ref, wq_ref, wo_ref, k_ref, v_ref, out_ref,
             send_ref, recv_ref, attn_ref, send_sems, recv_sems):
        my = lax.axis_index("i")
        peers = [jnp.bitwise_xor(my, 1), 3 - my]

        barrier_sem = pltpu.get_barrier_semaphore()
        for stage in range(2):
            pl.semaphore_signal(barrier_sem, inc=1, device_id=(peers[stage],),
                                device_id_type=pl.DeviceIdType.MESH)
        pl.semaphore_wait(barrier_sem, 2)

        wq = wq_ref[...].astype(jnp.bfloat16)
        wo = wo_ref[...].astype(jnp.bfloat16)

        def attention(b):
            xb = x_ref[b].astype(jnp.bfloat16)
            qb = jnp.dot(xb, wq, preferred_element_type=jnp.float32)
            qb = (qb * 0.125).astype(jnp.bfloat16)
            kb = k_ref[b].astype(jnp.bfloat16)
            vb = v_ref[b].astype(jnp.bfloat16)
            for h in range(HQ_LOC):
                q = qb[:, h * DH:(h + 1) * DH]
                k = kb[:, h * DH:(h + 1) * DH]
                v = vb[:, h * DH:(h + 1) * DH]
                s = lax.dot_general(q, k, (((1,), (1,)), ((), ())),
                                    preferred_element_type=jnp.float32)
                p = jnp.exp(s)
                denom = jnp.sum(p, axis=-1, keepdims=True)
                o = jnp.dot(p.astype(jnp.bfloat16), v,
                            preferred_element_type=jnp.float32)
                o = o * (1.0 / denom)
                attn_ref[:, h * DH:(h + 1) * DH] = o.astype(jnp.bfloat16)

        def exchange(slot, stage):
            rdma = pltpu.make_async_remote_copy(
                src_ref=send_ref.at[slot],
                dst_ref=recv_ref.at[slot],
                send_sem=send_sems.at[slot],
                recv_sem=recv_sems.at[slot],
                device_id=(peers[stage],),
                device_id_type=pl.DeviceIdType.MESH,
            )
            rdma.start()
            return rdma

        s0 = {}
        for b in range(B):
            attention(b)
            for dh in range(2):
                c = b * 2 + dh
                send_ref[c] = jnp.dot(
                    attn_ref[...], wo[:, dh * HD:(dh + 1) * HD],
                    preferred_element_type=jnp.float32,
                ).astype(jnp.bfloat16)
                s0[c] = exchange(c, 0)

        s1 = {}
        for c in range(NC):
            s0[c].wait()
            send_ref[NC + c] = send_ref[c] + recv_ref[c]
            s1[c] = exchange(NC + c, 1)

        for c in range(NC):
            b, dh = divmod(c, 2)
            s1[c].wait()
            out_ref[b, :, dh * HD:(dh + 1) * HD] = (
                send_ref[NC + c].astype(jnp.float32)
                + recv_ref[NC + c].astype(jnp.float32))

    K_t = K_ext.reshape(B, SKV, DQ_LOC)
    V_t = V_ext.reshape(B, SKV, DQ_LOC)
    return pl.pallas_call(
        body,
        out_shape=jax.ShapeDtypeStruct((B, SQ, D), jnp.float32),
        in_specs=[pl.BlockSpec(memory_space=pltpu.VMEM)] * 5,
        out_specs=pl.BlockSpec(memory_space=pltpu.VMEM),
        scratch_shapes=[
            pltpu.VMEM((2 * NC, SQ, HD), jnp.bfloat16),
            pltpu.VMEM((2 * NC, SQ, HD), jnp.bfloat16),
            pltpu.VMEM((SQ, DQ_LOC), jnp.bfloat16),
            pltpu.SemaphoreType.DMA((2 * NC,)),
            pltpu.SemaphoreType.DMA((2 * NC,)),
        ],
        compiler_params=pltpu.CompilerParams(collective_id=0),
    )(x, Wq, Wo, K_t, V_t)
